# baseline (speedup 1.0000x reference)
import functools

import jax
import jax.numpy as jnp
from jax import lax
from jax.experimental import pallas as pl
from jax.experimental.pallas import tpu as pltpu
from jax.experimental.pallas import tpu_sc as plsc

L = 32768
D = 1024
N = 16
BL = 1024
EPS = 1e-6

NSUB = 16
C = L // NSUB
VPC = C // 16


NSPLIT = 8
DS = D // NSPLIT


def _tc_body(*refs):
    x_refs = refs[:NSPLIT]
    p_ref, carry_ref, nrm_ref = refs[NSPLIT], refs[NSPLIT + 1], refs[NSPLIT + 2]
    i = pl.program_id(0)
    sumsq = jnp.zeros((BL, 1), jnp.float32)
    dot = jnp.zeros((BL, 1), jnp.float32)
    for k in range(NSPLIT):
        q = x_refs[k][:, :]
        sumsq = sumsq + jnp.sum(q * q, axis=1, keepdims=True)
        prev = jnp.where(i == 0, q[0:1, :],
                         carry_ref[0:1, k * DS:(k + 1) * DS])
        shifted = jnp.concatenate([prev, q[:-1, :]], axis=0)
        dot = dot + jnp.sum(q * shifted, axis=1, keepdims=True)
        carry_ref[0:1, k * DS:(k + 1) * DS] = q[BL - 1:BL, :]
    norm = jnp.sqrt(sumsq) + EPS
    nprev0 = jnp.where(i == 0, norm[0:1, :], nrm_ref[0:1, :])
    nprev = jnp.concatenate([nprev0, norm[:-1, :]], axis=0)
    cos = dot / (nprev * norm)
    p_ref[:, :] = jnp.clip(0.5 * (1.0 - cos), 0.0, 1.0)
    nrm_ref[0:1, :] = norm[BL - 1:BL, :]


def _tc_raw_p(x):
    return pl.pallas_call(
        _tc_body,
        grid=(L // BL,),
        in_specs=[
            pl.BlockSpec((BL, DS), functools.partial(
                lambda k, i: (i, k), k))
            for k in range(NSPLIT)
        ],
        out_specs=pl.BlockSpec((BL, 1), lambda i: (i, 0)),
        out_shape=jax.ShapeDtypeStruct((L, 1), jnp.float32),
        scratch_shapes=[pltpu.VMEM((8, D), jnp.float32),
                        pltpu.VMEM((8, 1), jnp.float32)],
    )(*([x] * NSPLIT))


def _sc_body(p_hbm, lens_hbm, mask_hbm, rp_hbm, sel_hbm, cu_hbm,
             p_v, lens_v, mask_v, rp_v, sel_v, cu_v, cnt_smem):
    wid = lax.axis_index("s")
    base = wid * C

    pltpu.sync_copy(p_hbm.at[pl.ds(base, C)], p_v)
    pltpu.sync_copy(lens_hbm, lens_v)

    lane = lax.broadcasted_iota(jnp.int32, (16,), 0)

    lv = lens_v[...]
    bpos = lax.cumsum(lv, axis=0) - lv
    inb = jnp.logical_and(bpos >= base, bpos < base + C)
    li = jnp.clip(bpos - base, 0, C - 1)
    plsc.store_scatter(p_v, [li], jnp.ones((16,), jnp.float32), mask=inb)

    @pl.when(wid == 0)
    def _():
        cnt_smem[0] = 0
    plsc.subcore_barrier()

    def body(j, cnt):
        pv = p_v[pl.ds(j * 16, 16)]
        q = 1.0 - pv
        m = pv > 0.5
        mask_v[pl.ds(j * 16, 16)] = m.astype(jnp.int32)
        sel_v[pl.ds(j * 16, 16)] = jnp.maximum(pv, q)
        idx2 = (j * 16 + lane) * 2
        plsc.store_scatter(rp_v, [idx2], q)
        plsc.store_scatter(rp_v, [idx2 + 1], pv)
        return cnt + m.astype(jnp.int32)

    cnt = lax.fori_loop(0, VPC, body,
                        jnp.zeros((16,), jnp.int32), unroll=4)
    local = lax.reduce_sum(cnt, axes=(0,))

    plsc.fetch_and_add(cnt_smem, local, subcore_id=0)
    plsc.subcore_barrier()

    pltpu.sync_copy(mask_v, mask_hbm.at[pl.ds(base, C)])
    pltpu.sync_copy(sel_v, sel_hbm.at[pl.ds(base, C)])
    pltpu.sync_copy(rp_v, rp_hbm.at[pl.ds(2 * base, 2 * C)])

    @pl.when(wid == 0)
    def _():
        total = jnp.broadcast_to(cnt_smem[0], (16,))
        cu_v[...] = jnp.where(lane == 1, total, 0)
        pltpu.sync_copy(cu_v, cu_hbm)


@functools.cache
def _get_sc_post():
    return functools.partial(
        pl.kernel,
        mesh=plsc.VectorSubcoreMesh(core_axis_name="c", subcore_axis_name="s",
                                    num_cores=1),
        compiler_params=pltpu.CompilerParams(needs_layout_passes=False),
        out_type=[
            jax.ShapeDtypeStruct((L,), jnp.int32),
            jax.ShapeDtypeStruct((2 * L,), jnp.float32),
            jax.ShapeDtypeStruct((L,), jnp.float32),
            jax.ShapeDtypeStruct((16,), jnp.int32),
        ],
        scratch_types=[
            pltpu.VMEM((C,), jnp.float32),
            pltpu.VMEM((16,), jnp.int32),
            pltpu.VMEM((C,), jnp.int32),
            pltpu.VMEM((2 * C,), jnp.float32),
            pltpu.VMEM((C,), jnp.float32),
            pltpu.VMEM((16,), jnp.int32),
            pltpu.SMEM((1,), jnp.int32),
        ],
    )(_sc_body)


@jax.jit
def kernel(hidden_states, x_pack_kwargs):
    x = hidden_states.reshape(L, D)
    p = _tc_raw_p(x).reshape(L)
    lens = x_pack_kwargs.reshape(N)
    mask_i, rp, sel, cu16 = _get_sc_post()(p, lens)
    token_mask = mask_i.astype(jnp.bool_).reshape(1, L)
    router_probs = rp.reshape(1, L, 2)
    selected_probs = sel.reshape(1, L, 1)
    return (token_mask, router_probs, selected_probs, cu16[:2])


# speedup vs baseline: 1.1973x; 1.1973x over previous
import functools

import jax
import jax.numpy as jnp
from jax import lax
from jax.experimental import pallas as pl
from jax.experimental.pallas import tpu as pltpu
from jax.experimental.pallas import tpu_sc as plsc

L = 32768
D = 1024
N = 16
BL = 2048
EPS = 1e-6

NSUB = 16
C = L // NSUB
VPC = C // 16


NSPLIT = 4
DS = D // NSPLIT


def _tc_body(*refs):
    x_refs = refs[:NSPLIT]
    p_ref, carry_ref, nrm_ref = refs[NSPLIT], refs[NSPLIT + 1], refs[NSPLIT + 2]
    i = pl.program_id(0)
    sumsq = jnp.zeros((BL, 1), jnp.float32)
    dot = jnp.zeros((BL, 1), jnp.float32)
    for k in range(NSPLIT):
        q = x_refs[k][:, :]
        sumsq = sumsq + jnp.sum(q * q, axis=1, keepdims=True)
        prev = jnp.where(i == 0, q[0:1, :],
                         carry_ref[0:1, k * DS:(k + 1) * DS])
        shifted = jnp.concatenate([prev, q[:-1, :]], axis=0)
        dot = dot + jnp.sum(q * shifted, axis=1, keepdims=True)
        carry_ref[0:1, k * DS:(k + 1) * DS] = q[BL - 1:BL, :]
    norm = jnp.sqrt(sumsq) + EPS
    nprev0 = jnp.where(i == 0, norm[0:1, :], nrm_ref[0:1, :])
    nprev = jnp.concatenate([nprev0, norm[:-1, :]], axis=0)
    cos = dot / (nprev * norm)
    p_ref[:, :] = jnp.clip(0.5 * (1.0 - cos), 0.0, 1.0)
    nrm_ref[0:1, :] = norm[BL - 1:BL, :]


def _tc_raw_p(x):
    return pl.pallas_call(
        _tc_body,
        grid=(L // BL,),
        in_specs=[
            pl.BlockSpec((BL, DS), functools.partial(
                lambda k, i: (i, k), k))
            for k in range(NSPLIT)
        ],
        out_specs=pl.BlockSpec((BL, 1), lambda i: (i, 0)),
        out_shape=jax.ShapeDtypeStruct((L, 1), jnp.float32),
        scratch_shapes=[pltpu.VMEM((8, D), jnp.float32),
                        pltpu.VMEM((8, 1), jnp.float32)],
    )(*([x] * NSPLIT))


def _sc_body(p_hbm, lens_hbm, mask_hbm, rp_hbm, sel_hbm, cu_hbm,
             p_v, lens_v, mask_v, rp_v, sel_v, cu_v, cnt_smem):
    wid = lax.axis_index("s")
    base = wid * C

    pltpu.sync_copy(p_hbm.at[pl.ds(base, C)], p_v)
    pltpu.sync_copy(lens_hbm, lens_v)

    lane = lax.broadcasted_iota(jnp.int32, (16,), 0)

    lv = lens_v[...]
    bpos = lax.cumsum(lv, axis=0) - lv
    inb = jnp.logical_and(bpos >= base, bpos < base + C)
    li = jnp.clip(bpos - base, 0, C - 1)
    plsc.store_scatter(p_v, [li], jnp.ones((16,), jnp.float32), mask=inb)

    @pl.when(wid == 0)
    def _():
        cnt_smem[0] = 0
    plsc.subcore_barrier()

    def body(j, cnt):
        pv = p_v[pl.ds(j * 16, 16)]
        q = 1.0 - pv
        m = pv > 0.5
        mask_v[pl.ds(j * 16, 16)] = m.astype(jnp.int32)
        sel_v[pl.ds(j * 16, 16)] = jnp.maximum(pv, q)
        idx2 = (j * 16 + lane) * 2
        plsc.store_scatter(rp_v, [idx2], q)
        plsc.store_scatter(rp_v, [idx2 + 1], pv)
        return cnt + m.astype(jnp.int32)

    cnt = lax.fori_loop(0, VPC, body,
                        jnp.zeros((16,), jnp.int32), unroll=4)
    local = lax.reduce_sum(cnt, axes=(0,))

    plsc.fetch_and_add(cnt_smem, local, subcore_id=0)
    plsc.subcore_barrier()

    pltpu.sync_copy(mask_v, mask_hbm.at[pl.ds(base, C)])
    pltpu.sync_copy(sel_v, sel_hbm.at[pl.ds(base, C)])
    pltpu.sync_copy(rp_v, rp_hbm.at[pl.ds(2 * base, 2 * C)])

    @pl.when(wid == 0)
    def _():
        total = jnp.broadcast_to(cnt_smem[0], (16,))
        cu_v[...] = jnp.where(lane == 1, total, 0)
        pltpu.sync_copy(cu_v, cu_hbm)


@functools.cache
def _get_sc_post():
    return functools.partial(
        pl.kernel,
        mesh=plsc.VectorSubcoreMesh(core_axis_name="c", subcore_axis_name="s",
                                    num_cores=1),
        compiler_params=pltpu.CompilerParams(needs_layout_passes=False),
        out_type=[
            jax.ShapeDtypeStruct((L,), jnp.int32),
            jax.ShapeDtypeStruct((2 * L,), jnp.float32),
            jax.ShapeDtypeStruct((L,), jnp.float32),
            jax.ShapeDtypeStruct((16,), jnp.int32),
        ],
        scratch_types=[
            pltpu.VMEM((C,), jnp.float32),
            pltpu.VMEM((16,), jnp.int32),
            pltpu.VMEM((C,), jnp.int32),
            pltpu.VMEM((2 * C,), jnp.float32),
            pltpu.VMEM((C,), jnp.float32),
            pltpu.VMEM((16,), jnp.int32),
            pltpu.SMEM((1,), jnp.int32),
        ],
    )(_sc_body)


@jax.jit
def kernel(hidden_states, x_pack_kwargs):
    x = hidden_states.reshape(L, D)
    p = _tc_raw_p(x).reshape(L)
    lens = x_pack_kwargs.reshape(N)
    mask_i, rp, sel, cu16 = _get_sc_post()(p, lens)
    token_mask = mask_i.astype(jnp.bool_).reshape(1, L)
    router_probs = rp.reshape(1, L, 2)
    selected_probs = sel.reshape(1, L, 1)
    return (token_mask, router_probs, selected_probs, cu16[:2])


# speedup vs baseline: 1.2204x; 1.0193x over previous
import functools

import jax
import jax.numpy as jnp
from jax import lax
from jax.experimental import pallas as pl
from jax.experimental.pallas import tpu as pltpu
from jax.experimental.pallas import tpu_sc as plsc

L = 32768
D = 1024
N = 16
BL = 4096
EPS = 1e-6

NSUB = 16
C = L // NSUB
VPC = C // 16


NSPLIT = 4
DS = D // NSPLIT


def _tc_body(*refs):
    x_refs = refs[:NSPLIT]
    p_ref, carry_ref, nrm_ref = refs[NSPLIT], refs[NSPLIT + 1], refs[NSPLIT + 2]
    i = pl.program_id(0)
    sumsq = jnp.zeros((BL, 1), jnp.float32)
    dot = jnp.zeros((BL, 1), jnp.float32)
    for k in range(NSPLIT):
        q = x_refs[k][:, :]
        sumsq = sumsq + jnp.sum(q * q, axis=1, keepdims=True)
        prev = jnp.where(i == 0, q[0:1, :],
                         carry_ref[0:1, k * DS:(k + 1) * DS])
        shifted = jnp.concatenate([prev, q[:-1, :]], axis=0)
        dot = dot + jnp.sum(q * shifted, axis=1, keepdims=True)
        carry_ref[0:1, k * DS:(k + 1) * DS] = q[BL - 1:BL, :]
    norm = jnp.sqrt(sumsq) + EPS
    nprev0 = jnp.where(i == 0, norm[0:1, :], nrm_ref[0:1, :])
    nprev = jnp.concatenate([nprev0, norm[:-1, :]], axis=0)
    cos = dot / (nprev * norm)
    p_ref[:, :] = jnp.clip(0.5 * (1.0 - cos), 0.0, 1.0)
    nrm_ref[0:1, :] = norm[BL - 1:BL, :]


def _tc_raw_p(x):
    return pl.pallas_call(
        _tc_body,
        grid=(L // BL,),
        in_specs=[
            pl.BlockSpec((BL, DS), functools.partial(
                lambda k, i: (i, k), k))
            for k in range(NSPLIT)
        ],
        out_specs=pl.BlockSpec((BL, 1), lambda i: (i, 0)),
        out_shape=jax.ShapeDtypeStruct((L, 1), jnp.float32),
        scratch_shapes=[pltpu.VMEM((8, D), jnp.float32),
                        pltpu.VMEM((8, 1), jnp.float32)],
    )(*([x] * NSPLIT))


def _sc_body(p_hbm, lens_hbm, mask_hbm, rp_hbm, sel_hbm, cu_hbm,
             p_v, lens_v, mask_v, rp_v, sel_v, cu_v, cnt_smem):
    wid = lax.axis_index("s")
    base = wid * C

    pltpu.sync_copy(p_hbm.at[pl.ds(base, C)], p_v)
    pltpu.sync_copy(lens_hbm, lens_v)

    lane = lax.broadcasted_iota(jnp.int32, (16,), 0)

    lv = lens_v[...]
    bpos = lax.cumsum(lv, axis=0) - lv
    inb = jnp.logical_and(bpos >= base, bpos < base + C)
    li = jnp.clip(bpos - base, 0, C - 1)
    plsc.store_scatter(p_v, [li], jnp.ones((16,), jnp.float32), mask=inb)

    @pl.when(wid == 0)
    def _():
        cnt_smem[0] = 0
    plsc.subcore_barrier()

    def body(j, cnt):
        pv = p_v[pl.ds(j * 16, 16)]
        q = 1.0 - pv
        m = pv > 0.5
        mask_v[pl.ds(j * 16, 16)] = m.astype(jnp.int32)
        sel_v[pl.ds(j * 16, 16)] = jnp.maximum(pv, q)
        idx2 = (j * 16 + lane) * 2
        plsc.store_scatter(rp_v, [idx2], q)
        plsc.store_scatter(rp_v, [idx2 + 1], pv)
        return cnt + m.astype(jnp.int32)

    cnt = lax.fori_loop(0, VPC, body,
                        jnp.zeros((16,), jnp.int32), unroll=4)
    local = lax.reduce_sum(cnt, axes=(0,))

    plsc.fetch_and_add(cnt_smem, local, subcore_id=0)
    plsc.subcore_barrier()

    pltpu.sync_copy(mask_v, mask_hbm.at[pl.ds(base, C)])
    pltpu.sync_copy(sel_v, sel_hbm.at[pl.ds(base, C)])
    pltpu.sync_copy(rp_v, rp_hbm.at[pl.ds(2 * base, 2 * C)])

    @pl.when(wid == 0)
    def _():
        total = jnp.broadcast_to(cnt_smem[0], (16,))
        cu_v[...] = jnp.where(lane == 1, total, 0)
        pltpu.sync_copy(cu_v, cu_hbm)


@functools.cache
def _get_sc_post():
    return functools.partial(
        pl.kernel,
        mesh=plsc.VectorSubcoreMesh(core_axis_name="c", subcore_axis_name="s",
                                    num_cores=1),
        compiler_params=pltpu.CompilerParams(needs_layout_passes=False),
        out_type=[
            jax.ShapeDtypeStruct((L,), jnp.int32),
            jax.ShapeDtypeStruct((2 * L,), jnp.float32),
            jax.ShapeDtypeStruct((L,), jnp.float32),
            jax.ShapeDtypeStruct((16,), jnp.int32),
        ],
        scratch_types=[
            pltpu.VMEM((C,), jnp.float32),
            pltpu.VMEM((16,), jnp.int32),
            pltpu.VMEM((C,), jnp.int32),
            pltpu.VMEM((2 * C,), jnp.float32),
            pltpu.VMEM((C,), jnp.float32),
            pltpu.VMEM((16,), jnp.int32),
            pltpu.SMEM((1,), jnp.int32),
        ],
    )(_sc_body)


@jax.jit
def kernel(hidden_states, x_pack_kwargs):
    x = hidden_states.reshape(L, D)
    p = _tc_raw_p(x).reshape(L)
    lens = x_pack_kwargs.reshape(N)
    mask_i, rp, sel, cu16 = _get_sc_post()(p, lens)
    token_mask = mask_i.astype(jnp.bool_).reshape(1, L)
    router_probs = rp.reshape(1, L, 2)
    selected_probs = sel.reshape(1, L, 1)
    return (token_mask, router_probs, selected_probs, cu16[:2])
